# baseline (speedup 1.0000x reference)
import functools

import jax
import jax.numpy as jnp
from jax import lax
from jax.experimental import pallas as pl
from jax.experimental.pallas import tpu as pltpu
from jax.experimental.pallas import tpu_sc as plsc

D = 1000
NC = 2
NS = 16
NW = NC * NS
PC = 16
WC = 32
NPW = 100
NG = 62


@jax.jit
def _gather_tiled(idxT_flat, table):
    mesh = plsc.VectorSubcoreMesh(
        core_axis_name="c", subcore_axis_name="s", num_cores=NC, num_subcores=NS
    )

    @functools.partial(
        pl.kernel,
        mesh=mesh,
        out_type=jax.ShapeDtypeStruct((50, 125, 8, 8, 128), jnp.float32),
        scratch_types=[
            pltpu.VMEM((NPW * PC,), jnp.int32),
            pltpu.VMEM((2, PC, D), jnp.float32),
            pltpu.VMEM((2, 125, 8, WC + 1), jnp.float32),
            [pltpu.SemaphoreType.DMA] * 2,
            [pltpu.SemaphoreType.DMA] * 2,
        ],
        compiler_params=pltpu.CompilerParams(
            use_tc_tiling_on_sc=False, needs_layout_passes=False
        ),
    )
    def k(idx_hbm, table_hbm, out_hbm, idx_v, gbuf, tbuf, gsems, wsems):
        w = lax.axis_index("s") * NC + lax.axis_index("c")
        w0 = NPW // 2 * w
        pltpu.sync_copy(idx_hbm.at[pl.ds(NPW * PC * w, NPW * PC)], idx_v)

        iota = jnp.arange(16, dtype=jnp.int32)

        def gather_cp(P, s):
            return pltpu.make_async_copy(
                table_hbm.at[idx_v.at[pl.ds(PC * P, PC)]], gbuf.at[s], gsems[s]
            )

        def write_cp(W, ts):
            wg = w0 + W
            u = lax.shift_right_logical(wg, 2)
            p4 = lax.bitwise_and(wg, 3)
            t = lax.shift_right_logical(u, 3)
            j = lax.bitwise_and(u, 7)
            return pltpu.make_async_copy(
                tbuf.at[ts, :, :, pl.ds(0, WC)],
                out_hbm.at[t, :, j, :, pl.ds(WC * p4, WC)],
                wsems[ts],
            )

        def tr_group(src, dst, d0, coff):
            dvec = iota + d0
            rvec = lax.shift_right_logical(dvec, 3)
            svec = lax.bitwise_and(dvec, 7)
            for c in range(PC):
                vals = src[c, pl.ds(d0, 16)]
                cv = jnp.full((16,), coff + c, dtype=jnp.int32)
                plsc.store_scatter(dst, [rvec, svec, cv], vals)

        def transpose(gs, ts, coff):
            src = gbuf.at[gs]
            dst = tbuf.at[ts]

            @plsc.parallel_loop(0, NG, unroll=3)
            def tr_body(g):
                tr_group(src, dst, 16 * g, coff)

            tr_group(src, dst, D - 16, coff)

        def pair(W, ts, first, last):
            if not first:
                write_cp(W - 2, ts).wait()
            for sub in range(2):
                P = 2 * W + sub
                gather_cp(P, sub).wait()
                transpose(sub, ts, PC * sub)
                if not last:
                    gather_cp(P + 2, sub).start()
            write_cp(W, ts).start()

        gather_cp(0, 0).start()
        gather_cp(1, 1).start()
        pair(0, 0, True, False)
        pair(1, 1, True, False)

        def body(w2, carry):
            pair(2 * w2, 0, False, False)
            pair(2 * w2 + 1, 1, False, False)
            return carry

        lax.fori_loop(1, NPW // 4 - 1, body, 0)

        last = NPW // 2 - 2
        pair(last, 0, False, False)
        pair(last + 1, 1, False, True)
        write_cp(last, 0).wait()
        write_cp(last + 1, 1).wait()

    return k(idxT_flat, table)


def kernel(idx, table):
    b, t = idx.shape
    idxT_flat = idx.T.reshape(-1)
    p5 = _gather_tiled(idxT_flat, table)
    b5 = jnp.transpose(p5, (2, 4, 0, 1, 3))
    return b5.reshape(b, t, D)


# speedup vs baseline: 1.4119x; 1.4119x over previous
import functools

import jax
import jax.numpy as jnp
from jax import lax
from jax.experimental import pallas as pl
from jax.experimental.pallas import tpu as pltpu
from jax.experimental.pallas import tpu_sc as plsc

D = 1000
NC = 2
NS = 16
NW = NC * NS
PC = 16
WC = 32
NPW = 100
NG = 62


@jax.jit
def _gather_tiled(idxT_flat, table):
    mesh = plsc.VectorSubcoreMesh(
        core_axis_name="c", subcore_axis_name="s", num_cores=NC, num_subcores=NS
    )

    @functools.partial(
        pl.kernel,
        mesh=mesh,
        out_type=jax.ShapeDtypeStruct((50, 125, 8, 8, 128), jnp.float32),
        scratch_types=[
            pltpu.VMEM((NPW * PC,), jnp.int32),
            pltpu.VMEM((2, PC, D), jnp.float32),
            pltpu.VMEM((2, 125, 8, WC + 1), jnp.float32),
            [pltpu.SemaphoreType.DMA] * 2,
            [pltpu.SemaphoreType.DMA] * 2,
        ],
        compiler_params=pltpu.CompilerParams(
            use_tc_tiling_on_sc=False, needs_layout_passes=False
        ),
    )
    def k(idx_hbm, table_hbm, out_hbm, idx_v, gbuf, tbuf, gsems, wsems):
        w = lax.axis_index("s") * NC + lax.axis_index("c")
        w0 = NPW // 2 * w
        pltpu.sync_copy(idx_hbm.at[pl.ds(NPW * PC * w, NPW * PC)], idx_v)

        iota = jnp.arange(16, dtype=jnp.int32)

        def gather_cp(P, s):
            return pltpu.make_async_copy(
                table_hbm.at[idx_v.at[pl.ds(PC * P, PC)]], gbuf.at[s], gsems[s]
            )

        def write_cp(W, ts):
            wg = w0 + W
            u = lax.shift_right_logical(wg, 2)
            p4 = lax.bitwise_and(wg, 3)
            t = lax.shift_right_logical(u, 3)
            j = lax.bitwise_and(u, 7)
            return pltpu.make_async_copy(
                tbuf.at[ts, :, :, pl.ds(0, WC)],
                out_hbm.at[t, :, j, :, pl.ds(WC * p4, WC)],
                wsems[ts],
            )

        def tr_group(src, dst, d0, coff):
            dvec = iota + d0
            rvec = lax.shift_right_logical(dvec, 3)
            svec = lax.bitwise_and(dvec, 7)
            for c in range(PC):
                vals = src[c, pl.ds(d0, 16)]
                cv = jnp.full((16,), coff + c, dtype=jnp.int32)
                plsc.store_scatter(dst, [rvec, svec, cv], vals)

        def transpose(gs, ts, coff):
            src = gbuf.at[gs]
            dst = tbuf.at[ts]

            @plsc.parallel_loop(0, NG, unroll=2)
            def tr_body(g):
                tr_group(src, dst, 16 * g, coff)

            tr_group(src, dst, D - 16, coff)

        def pair(W, ts, first, last):
            if not first:
                write_cp(W - 2, ts).wait()
            for sub in range(2):
                P = 2 * W + sub
                gather_cp(P, sub).wait()
                transpose(sub, ts, PC * sub)
                if not last:
                    gather_cp(P + 2, sub).start()
            write_cp(W, ts).start()

        gather_cp(0, 0).start()
        gather_cp(1, 1).start()
        pair(0, 0, True, False)
        pair(1, 1, True, False)

        def body(w2, carry):
            pair(2 * w2, 0, False, False)
            pair(2 * w2 + 1, 1, False, False)
            return carry

        lax.fori_loop(1, NPW // 4 - 1, body, 0)

        last = NPW // 2 - 2
        pair(last, 0, False, False)
        pair(last + 1, 1, False, True)
        write_cp(last, 0).wait()
        write_cp(last + 1, 1).wait()

    return k(idxT_flat, table)


def kernel(idx, table):
    b, t = idx.shape
    idxT_flat = idx.T.reshape(-1)
    p5 = _gather_tiled(idxT_flat, table)
    b5 = jnp.transpose(p5, (2, 4, 0, 1, 3))
    return b5.reshape(b, t, D)
